# baseline (speedup 1.0000x reference)
import functools

import jax
import jax.numpy as jnp
from jax import lax
from jax.experimental import pallas as pl
from jax.experimental.pallas import tpu as pltpu
from jax.experimental.pallas import tpu_sc as plsc

N = 10000
D = 128
E = 160000
NC = 2
NS = 16
NPAD = 10240
RPT = NPAD // NS
EPT = 10240
EPAD = EPT * NS
AB = 80
ANB = EPT // AB
NQ = 2
CB = ANB // NQ
RB = 1000

_mesh = plsc.VectorSubcoreMesh(
    core_axis_name="c", subcore_axis_name="s", num_cores=NC, num_subcores=NS
)


def _deg_sc(dst_t):
    @functools.partial(
        pl.kernel,
        out_type=jax.ShapeDtypeStruct((NC, NPAD), jnp.float32),
        mesh=_mesh,
        scratch_types=[
            pltpu.VMEM_SHARED((NPAD,), jnp.float32),
            pltpu.VMEM((CB, AB), jnp.int32),
            pltpu.VMEM((AB,), jnp.float32),
            pltpu.VMEM((RPT,), jnp.float32),
        ],
    )
    def run(dst_ref, deg_ref, deg_sh, idx_v, ones_v, stage_v):
        c = lax.axis_index("c")
        s = lax.axis_index("s")
        for i in range(AB // 16):
            ones_v[pl.ds(i * 16, 16)] = jnp.ones((16,), jnp.float32)

        def zfill(i, carry):
            stage_v[pl.ds(i * 16, 16)] = jnp.zeros((16,), jnp.float32)
            return carry

        lax.fori_loop(0, RPT // 16, zfill, 0)
        pltpu.sync_copy(stage_v, deg_sh.at[pl.ds(s * RPT, RPT)])
        plsc.subcore_barrier()

        def body(j, carry):
            pltpu.sync_copy(ones_v, deg_sh.at[idx_v.at[j]], add=True)
            return carry

        for q in range(NQ):
            pltpu.sync_copy(dst_ref.at[c, s, pl.ds(q * CB, CB)], idx_v)
            lax.fori_loop(0, CB, body, 0)
        plsc.subcore_barrier()
        pltpu.sync_copy(deg_sh.at[pl.ds(s * RPT, RPT)], stage_v)
        pltpu.sync_copy(stage_v, deg_ref.at[c, pl.ds(s * RPT, RPT)])

    return run(dst_t)


def _agg_sc(src_t, dst_t, h2):
    @functools.partial(
        pl.kernel,
        out_type=jax.ShapeDtypeStruct((NC, NPAD, D), jnp.float32),
        mesh=_mesh,
        scratch_types=[
            pltpu.VMEM_SHARED((NPAD, D), jnp.float32),
            pltpu.VMEM((CB, AB), jnp.int32),
            pltpu.VMEM((CB, AB), jnp.int32),
            pltpu.VMEM((AB, D), jnp.float32),
            pltpu.VMEM((AB, D), jnp.float32),
            pltpu.SemaphoreType.DMA,
            pltpu.SemaphoreType.DMA,
            pltpu.SemaphoreType.DMA,
            pltpu.SemaphoreType.DMA,
        ],
    )
    def run(src_ref, dst_ref, h2_ref, agg_ref, acc_sh, sidx_v, didx_v,
            rows0, rows1, sem_g0, sem_g1, sem_s0, sem_s1):
        c = lax.axis_index("c")
        s = lax.axis_index("s")

        def zfill(r, carry):
            for k in range(D // 16):
                rows0[r, pl.ds(k * 16, 16)] = jnp.zeros((16,), jnp.float32)
            return carry

        lax.fori_loop(0, AB, zfill, 0)
        for t in range(RPT // AB):
            pltpu.sync_copy(rows0, acc_sh.at[pl.ds(s * RPT + t * AB, AB)])
        plsc.subcore_barrier()

        def scat(j, buf):
            pltpu.sync_copy(buf, acc_sh.at[didx_v.at[j]], add=True)

        def body(k, carry):
            m = 2 * k
            d0 = pltpu.async_copy(h2_ref.at[sidx_v.at[m]], rows0, sem_g0)
            d1 = pltpu.async_copy(h2_ref.at[sidx_v.at[m + 1]], rows1, sem_g1)
            d0.wait()
            scat(m, rows0)
            d1.wait()
            scat(m + 1, rows1)
            return carry

        for q in range(NQ):
            pltpu.sync_copy(src_ref.at[c, s, pl.ds(q * CB, CB)], sidx_v)
            pltpu.sync_copy(dst_ref.at[c, s, pl.ds(q * CB, CB)], didx_v)
            lax.fori_loop(0, CB // 2, body, 0)
        plsc.subcore_barrier()

        bufs = (rows0, rows1)
        sems = (sem_s0, sem_s1)
        nout = RPT // AB
        for t in range(nout):
            b = t % 2
            if t >= 2:
                pltpu.make_async_copy(
                    bufs[b], agg_ref.at[c, pl.ds(s * RPT + (t - 2) * AB, AB)], sems[b]
                ).wait()
            pltpu.sync_copy(acc_sh.at[pl.ds(s * RPT + t * AB, AB)], bufs[b])
            pltpu.async_copy(
                bufs[b], agg_ref.at[c, pl.ds(s * RPT + t * AB, AB)], sems[b]
            )
        for t in (nout - 2, nout - 1):
            b = t % 2
            pltpu.make_async_copy(
                bufs[b], agg_ref.at[c, pl.ds(s * RPT + t * AB, AB)], sems[b]
            ).wait()

    return run(src_t, dst_t, h2)


def _h2_tc(xp, W, deg):
    rb = 1024

    def body(x_ref, w_ref, deg_ref, h2_ref):
        h = jnp.dot(x_ref[...], w_ref[...], preferred_element_type=jnp.float32)
        dinv = lax.rsqrt(deg_ref[...] + 1.0)
        h2_ref[...] = h[None, :, :] * dinv

    return pl.pallas_call(
        body,
        grid=(NPAD // rb,),
        in_specs=[
            pl.BlockSpec((rb, D), lambda i: (i, 0)),
            pl.BlockSpec((D, D), lambda i: (0, 0)),
            pl.BlockSpec((NC, rb, 1), lambda i: (0, i, 0)),
        ],
        out_specs=pl.BlockSpec((NC, rb, D), lambda i: (0, i, 0)),
        out_shape=jax.ShapeDtypeStruct((NC, NPAD, D), jnp.float32),
    )(xp, W, deg)


def _final_tc(agg, h2, deg, bias):
    def body(agg_ref, h2_ref, deg_ref, b_ref, oa_ref, ob_ref):
        dinv = lax.rsqrt(deg_ref[...] + 1.0)
        r = dinv * (agg_ref[...] + h2_ref[...]) + b_ref[...][None]
        r = jnp.maximum(r, 0.0)
        oa_ref[...] = r[0]
        ob_ref[...] = r[1]

    return pl.pallas_call(
        body,
        grid=(N // RB,),
        in_specs=[
            pl.BlockSpec((NC, RB, D), lambda i: (0, i, 0)),
            pl.BlockSpec((NC, RB, D), lambda i: (0, i, 0)),
            pl.BlockSpec((NC, RB, 1), lambda i: (0, i, 0)),
            pl.BlockSpec((1, D), lambda i: (0, 0)),
        ],
        out_specs=[
            pl.BlockSpec((RB, D), lambda i: (i, 0)),
            pl.BlockSpec((RB, D), lambda i: (i, 0)),
        ],
        out_shape=[
            jax.ShapeDtypeStruct((N, D), jnp.float32),
            jax.ShapeDtypeStruct((N, D), jnp.float32),
        ],
    )(agg, h2, deg, bias)


def kernel(view_a_pos, view_a_neg, view_b_pos, view_b_neg, x, W, b):
    sa = view_a_pos[0].astype(jnp.int32)
    da = view_a_pos[1].astype(jnp.int32)
    sb = view_b_pos[0].astype(jnp.int32)
    db = view_b_pos[1].astype(jnp.int32)
    npd = jnp.arange(EPAD - E, dtype=jnp.int32)
    pad_src = jnp.full((EPAD - E,), N, jnp.int32)
    pad_dst_agg = npd % N
    pad_dst_deg = N + (npd % (NPAD - N))
    src_p = jnp.stack(
        [jnp.concatenate([sa, pad_src]), jnp.concatenate([sb + NPAD, pad_src])]
    ).reshape(NC, NS, ANB, AB)
    dst_agg = jnp.stack(
        [jnp.concatenate([da, pad_dst_agg]), jnp.concatenate([db, pad_dst_agg])]
    ).reshape(NC, NS, ANB, AB)
    dst_deg = jnp.stack(
        [jnp.concatenate([da, pad_dst_deg]), jnp.concatenate([db, pad_dst_deg])]
    ).reshape(NC, NS, ANB, AB)

    xp = jnp.concatenate([x, jnp.zeros((NPAD - N, D), jnp.float32)])
    deg = _deg_sc(dst_deg).reshape(NC, NPAD, 1)
    h2 = _h2_tc(xp, W, deg)
    agg = _agg_sc(src_p, dst_agg, h2.reshape(NC * NPAD, D))
    xa, xb = _final_tc(agg, h2, deg, b.reshape(1, D))
    return (xa, xb)


# speedup vs baseline: 1.7663x; 1.7663x over previous
import functools

import jax
import jax.numpy as jnp
from jax import lax
from jax.experimental import pallas as pl
from jax.experimental.pallas import tpu as pltpu
from jax.experimental.pallas import tpu_sc as plsc

N = 10000
D = 128
E = 160000
NC = 2
NS = 16
NPAD = 10240
RPT = NPAD // NS
EPT = 10240
EPAD = EPT * NS
AB = 80
ANB = EPT // AB
NQ = 2
CB = ANB // NQ
RB = 1000

_mesh = plsc.VectorSubcoreMesh(
    core_axis_name="c", subcore_axis_name="s", num_cores=NC, num_subcores=NS
)


def _deg_sc(dst_t):
    @functools.partial(
        pl.kernel,
        out_type=jax.ShapeDtypeStruct((NC, NPAD), jnp.float32),
        mesh=_mesh,
        scratch_types=[
            pltpu.VMEM_SHARED((NPAD,), jnp.float32),
            pltpu.VMEM((CB, AB), jnp.int32),
            pltpu.VMEM((AB,), jnp.float32),
            pltpu.VMEM((RPT,), jnp.float32),
        ],
    )
    def run(dst_ref, deg_ref, deg_sh, idx_v, ones_v, stage_v):
        c = lax.axis_index("c")
        s = lax.axis_index("s")
        for i in range(AB // 16):
            ones_v[pl.ds(i * 16, 16)] = jnp.ones((16,), jnp.float32)

        def zfill(i, carry):
            stage_v[pl.ds(i * 16, 16)] = jnp.zeros((16,), jnp.float32)
            return carry

        lax.fori_loop(0, RPT // 16, zfill, 0)
        pltpu.sync_copy(stage_v, deg_sh.at[pl.ds(s * RPT, RPT)])
        plsc.subcore_barrier()

        def body(j, carry):
            pltpu.sync_copy(ones_v, deg_sh.at[idx_v.at[j]], add=True)
            return carry

        for q in range(NQ):
            pltpu.sync_copy(dst_ref.at[c, s, pl.ds(q * CB, CB)], idx_v)
            lax.fori_loop(0, CB, body, 0)
        plsc.subcore_barrier()
        pltpu.sync_copy(deg_sh.at[pl.ds(s * RPT, RPT)], stage_v)
        pltpu.sync_copy(stage_v, deg_ref.at[c, pl.ds(s * RPT, RPT)])

    return run(dst_t)


def _agg_sc(src_t, dst_t, h2):
    @functools.partial(
        pl.kernel,
        out_type=jax.ShapeDtypeStruct((NC, NPAD, D), jnp.float32),
        mesh=_mesh,
        scratch_types=[
            pltpu.VMEM_SHARED((NPAD, D), jnp.float32),
            pltpu.VMEM((125, 80), jnp.int32),
            pltpu.VMEM((125, 80), jnp.int32),
            pltpu.VMEM((80, D), jnp.float32),
            pltpu.SemaphoreType.DMA,
        ],
    )
    def run(src_ref, dst_ref, h2_ref, agg_ref, acc_sh, sidx_v, didx_v, rows_v, sem):
        c = lax.axis_index("c")
        s = lax.axis_index("s")

        def zfill(r, carry):
            for k in range(D // 16):
                rows_v[r, pl.ds(k * 16, 16)] = jnp.zeros((16,), jnp.float32)
            return carry

        lax.fori_loop(0, 80, zfill, 0)
        for t in range(RPT // 80):
            pltpu.sync_copy(rows_v, acc_sh.at[pl.ds(s * RPT + t * 80, 80)])
        pltpu.sync_copy(src_ref.at[c, s], sidx_v)
        pltpu.sync_copy(dst_ref.at[c, s], didx_v)
        plsc.subcore_barrier()

        def body(j, carry):
            pltpu.async_copy(h2_ref.at[sidx_v.at[j]], rows_v, sem).wait()
            pltpu.sync_copy(rows_v, acc_sh.at[didx_v.at[j]], add=True)
            return carry

        lax.fori_loop(0, 125, body, 0)
        plsc.subcore_barrier()
        for t in range(RPT // 80):
            pltpu.sync_copy(acc_sh.at[pl.ds(s * RPT + t * 80, 80)], rows_v)
            pltpu.sync_copy(rows_v, agg_ref.at[c, pl.ds(s * RPT + t * 80, 80)])

    return run(src_t, dst_t, h2)


def _h2_tc(x, W, deg):
    def body(x_ref, w_ref, deg_ref, h2_ref):
        h = jnp.dot(x_ref[...], w_ref[...], preferred_element_type=jnp.float32)
        dinv = lax.rsqrt(deg_ref[...] + 1.0)
        h2_ref[...] = h[None, :, :] * dinv

    return pl.pallas_call(
        body,
        grid=(N // RB,),
        in_specs=[
            pl.BlockSpec((RB, D), lambda i: (i, 0)),
            pl.BlockSpec((D, D), lambda i: (0, 0)),
            pl.BlockSpec((NC, RB, 1), lambda i: (0, i, 0)),
        ],
        out_specs=pl.BlockSpec((NC, RB, D), lambda i: (0, i, 0)),
        out_shape=jax.ShapeDtypeStruct((NC, N, D), jnp.float32),
    )(x, W, deg)


def _final_tc(agg, h2, deg, bias):
    def body(agg_ref, h2_ref, deg_ref, b_ref, oa_ref, ob_ref):
        dinv = lax.rsqrt(deg_ref[...] + 1.0)
        r = dinv * (agg_ref[...] + h2_ref[...]) + b_ref[...][None]
        r = jnp.maximum(r, 0.0)
        oa_ref[...] = r[0]
        ob_ref[...] = r[1]

    return pl.pallas_call(
        body,
        grid=(N // RB,),
        in_specs=[
            pl.BlockSpec((NC, RB, D), lambda i: (0, i, 0)),
            pl.BlockSpec((NC, RB, D), lambda i: (0, i, 0)),
            pl.BlockSpec((NC, RB, 1), lambda i: (0, i, 0)),
            pl.BlockSpec((1, D), lambda i: (0, 0)),
        ],
        out_specs=[
            pl.BlockSpec((RB, D), lambda i: (i, 0)),
            pl.BlockSpec((RB, D), lambda i: (i, 0)),
        ],
        out_shape=[
            jax.ShapeDtypeStruct((N, D), jnp.float32),
            jax.ShapeDtypeStruct((N, D), jnp.float32),
        ],
    )(agg, h2, deg, bias)


def kernel(view_a_pos, view_a_neg, view_b_pos, view_b_neg, x, W, b):
    sa = view_a_pos[0].astype(jnp.int32)
    da = view_a_pos[1].astype(jnp.int32)
    sb = view_b_pos[0].astype(jnp.int32)
    db = view_b_pos[1].astype(jnp.int32)
    npd = jnp.arange(EPAD - E, dtype=jnp.int32)
    pad_dst_deg = N + (npd % (NPAD - N))
    dst_deg = jnp.stack(
        [jnp.concatenate([da, pad_dst_deg]), jnp.concatenate([db, pad_dst_deg])]
    ).reshape(NC, NS, ANB, AB)
    src_t = jnp.stack([sa, sb + N]).reshape(NC, NS, 125, 80)
    dst_t = jnp.stack([da, db]).reshape(NC, NS, 125, 80)

    deg = _deg_sc(dst_deg).reshape(NC, NPAD, 1)
    h2 = _h2_tc(x, W, deg)
    agg = _agg_sc(src_t, dst_t, h2.reshape(NC * N, D))
    xa, xb = _final_tc(agg, h2, deg, b.reshape(1, D))
    return (xa, xb)


# speedup vs baseline: 1.9418x; 1.0993x over previous
import functools

import jax
import jax.numpy as jnp
from jax import lax
from jax.experimental import pallas as pl
from jax.experimental.pallas import tpu as pltpu
from jax.experimental.pallas import tpu_sc as plsc

N = 10000
D = 128
E = 160000
NC = 2
NS = 16
NPAD = 10240
RPT = NPAD // NS
EPT = 10240
EPAD = EPT * NS
AB = 80
ANB = EPT // AB
NQ = 2
CB = ANB // NQ
RB = 1000
TROWS = 24000

_mesh = plsc.VectorSubcoreMesh(
    core_axis_name="c", subcore_axis_name="s", num_cores=NC, num_subcores=NS
)


def _deg_sc(dst_t):
    @functools.partial(
        pl.kernel,
        out_type=jax.ShapeDtypeStruct((NC, NPAD), jnp.float32),
        mesh=_mesh,
        scratch_types=[
            pltpu.VMEM_SHARED((NPAD,), jnp.float32),
            pltpu.VMEM((CB, AB), jnp.int32),
            pltpu.VMEM((AB,), jnp.float32),
            pltpu.VMEM((RPT,), jnp.float32),
        ],
    )
    def run(dst_ref, deg_ref, deg_sh, idx_v, ones_v, stage_v):
        c = lax.axis_index("c")
        s = lax.axis_index("s")
        for i in range(AB // 16):
            ones_v[pl.ds(i * 16, 16)] = jnp.ones((16,), jnp.float32)

        def zfill(i, carry):
            stage_v[pl.ds(i * 16, 16)] = jnp.zeros((16,), jnp.float32)
            return carry

        lax.fori_loop(0, RPT // 16, zfill, 0)
        pltpu.sync_copy(stage_v, deg_sh.at[pl.ds(s * RPT, RPT)])
        plsc.subcore_barrier()

        def body(j, carry):
            pltpu.sync_copy(ones_v, deg_sh.at[idx_v.at[j]], add=True)
            return carry

        for q in range(NQ):
            pltpu.sync_copy(dst_ref.at[c, s, pl.ds(q * CB, CB)], idx_v)
            lax.fori_loop(0, CB, body, 0)
        plsc.subcore_barrier()
        pltpu.sync_copy(deg_sh.at[pl.ds(s * RPT, RPT)], stage_v)
        pltpu.sync_copy(stage_v, deg_ref.at[c, pl.ds(s * RPT, RPT)])

    return run(dst_t)


def _agg_sc(src_t, dst_t, h2):
    @functools.partial(
        pl.kernel,
        out_type=jax.ShapeDtypeStruct((NC, NPAD, D), jnp.float32),
        mesh=_mesh,
        scratch_types=[
            pltpu.VMEM_SHARED((NPAD, D), jnp.float32),
            pltpu.VMEM((CB, AB), jnp.int32),
            pltpu.VMEM((CB, AB), jnp.int32),
            pltpu.VMEM((AB, D), jnp.float32),
            pltpu.VMEM((AB, D), jnp.float32),
            pltpu.SemaphoreType.DMA,
            pltpu.SemaphoreType.DMA,
        ],
    )
    def run(src_ref, dst_ref, h2_ref, agg_ref, acc_sh, sidx_v, didx_v,
            rows0, rows1, sem_g0, sem_g1):
        c = lax.axis_index("c")
        s = lax.axis_index("s")

        def zfill(r, carry):
            for k in range(D // 16):
                rows0[r, pl.ds(k * 16, 16)] = jnp.zeros((16,), jnp.float32)
            return carry

        lax.fori_loop(0, AB, zfill, 0)
        for t in range(RPT // AB):
            pltpu.sync_copy(rows0, acc_sh.at[pl.ds(s * RPT + t * AB, AB)])
        plsc.subcore_barrier()

        def body(k, carry):
            m = 2 * k
            d0 = pltpu.async_copy(h2_ref.at[sidx_v.at[m]], rows0, sem_g0)
            d1 = pltpu.async_copy(h2_ref.at[sidx_v.at[m + 1]], rows1, sem_g1)
            d0.wait()
            pltpu.sync_copy(rows0, acc_sh.at[didx_v.at[m]], add=True)
            d1.wait()
            pltpu.sync_copy(rows1, acc_sh.at[didx_v.at[m + 1]], add=True)
            return carry

        for q in range(NQ):
            pltpu.sync_copy(src_ref.at[c, s, pl.ds(q * CB, CB)], sidx_v)
            pltpu.sync_copy(dst_ref.at[c, s, pl.ds(q * CB, CB)], didx_v)
            lax.fori_loop(0, CB // 2, body, 0)
        plsc.subcore_barrier()
        for t in range(RPT // AB):
            pltpu.sync_copy(acc_sh.at[pl.ds(s * RPT + t * AB, AB)], rows0)
            pltpu.sync_copy(rows0, agg_ref.at[c, pl.ds(s * RPT + t * AB, AB)])

    return run(src_t, dst_t, h2)


def _h2_tc(x, W, deg):
    ng = TROWS // RB

    def body(x_ref, w_ref, deg_ref, h2_ref):
        i = pl.program_id(0)
        h = jnp.dot(x_ref[...], w_ref[...], preferred_element_type=jnp.float32)
        dinv = lax.rsqrt(deg_ref[...] + 1.0)
        h2_ref[...] = jnp.where(i < NC * (N // RB), h * dinv[0], 0.0)

    return pl.pallas_call(
        body,
        grid=(ng,),
        in_specs=[
            pl.BlockSpec((RB, D), lambda i: (i % (N // RB), 0)),
            pl.BlockSpec((D, D), lambda i: (0, 0)),
            pl.BlockSpec(
                (1, RB, 1),
                lambda i: (jnp.minimum(i // (N // RB), NC - 1), i % (N // RB), 0),
            ),
        ],
        out_specs=pl.BlockSpec((RB, D), lambda i: (i, 0)),
        out_shape=jax.ShapeDtypeStruct((TROWS, D), jnp.float32),
    )(x, W, deg)


def _final_tc(agg, h2, deg, bias):
    def body(agg_ref, ha_ref, hb_ref, deg_ref, b_ref, oa_ref, ob_ref):
        dinv = lax.rsqrt(deg_ref[...] + 1.0)
        h = jnp.stack([ha_ref[...], hb_ref[...]])
        r = dinv * (agg_ref[...] + h) + b_ref[...][None]
        r = jnp.maximum(r, 0.0)
        oa_ref[...] = r[0]
        ob_ref[...] = r[1]

    return pl.pallas_call(
        body,
        grid=(N // RB,),
        in_specs=[
            pl.BlockSpec((NC, RB, D), lambda i: (0, i, 0)),
            pl.BlockSpec((RB, D), lambda i: (i, 0)),
            pl.BlockSpec((RB, D), lambda i: (i + N // RB, 0)),
            pl.BlockSpec((NC, RB, 1), lambda i: (0, i, 0)),
            pl.BlockSpec((1, D), lambda i: (0, 0)),
        ],
        out_specs=[
            pl.BlockSpec((RB, D), lambda i: (i, 0)),
            pl.BlockSpec((RB, D), lambda i: (i, 0)),
        ],
        out_shape=[
            jax.ShapeDtypeStruct((N, D), jnp.float32),
            jax.ShapeDtypeStruct((N, D), jnp.float32),
        ],
    )(agg, h2, h2, deg, bias)


def kernel(view_a_pos, view_a_neg, view_b_pos, view_b_neg, x, W, b):
    sa = view_a_pos[0].astype(jnp.int32)
    da = view_a_pos[1].astype(jnp.int32)
    sb = view_b_pos[0].astype(jnp.int32)
    db = view_b_pos[1].astype(jnp.int32)
    npd = jnp.arange(EPAD - E, dtype=jnp.int32)
    pad_src = NC * N + npd
    pad_dst_agg = npd % N
    pad_dst_deg = N + (npd % (NPAD - N))
    src_p = jnp.stack(
        [jnp.concatenate([sa, pad_src]), jnp.concatenate([sb + N, pad_src])]
    ).reshape(NC, NS, ANB, AB)
    dst_agg = jnp.stack(
        [jnp.concatenate([da, pad_dst_agg]), jnp.concatenate([db, pad_dst_agg])]
    ).reshape(NC, NS, ANB, AB)
    dst_deg = jnp.stack(
        [jnp.concatenate([da, pad_dst_deg]), jnp.concatenate([db, pad_dst_deg])]
    ).reshape(NC, NS, ANB, AB)

    deg = _deg_sc(dst_deg).reshape(NC, NPAD, 1)
    h2 = _h2_tc(x, W, deg)
    agg = _agg_sc(src_p, dst_agg, h2)
    xa, xb = _final_tc(agg, h2, deg, b.reshape(1, D))
    return (xa, xb)
